# baseline (speedup 1.0000x reference)
import functools
import jax
import jax.numpy as jnp
import numpy as np
from jax.experimental import pallas as pl
from jax.experimental.pallas import tpu as pltpu

N = 10000
E = 160000
H = 128
NH = 8
DH = 16
NG = 64
FF = 256
NL = 4

BN_E = 1000
BN_N = 1000


def _dot(a, b):
    return jax.lax.dot_general(a, b, (((1,), (0,)), ((), ())),
                               precision=jax.lax.Precision.HIGHEST,
                               preferred_element_type=jnp.float32)


def _dotd(a, b):
    return jax.lax.dot_general(a, b, (((1,), (0,)), ((), ())),
                               preferred_element_type=jnp.float32)


def _emb_body(x_ref, pe_ref, w1_ref, w2_ref, o_ref):
    o_ref[...] = (_dotd(x_ref[...], w1_ref[...]) +
                  _dotd(pe_ref[...], w2_ref[...]))


def _emb(x, pe, w1, w2):
    g = N // BN_N
    return pl.pallas_call(
        _emb_body,
        grid=(g,),
        in_specs=[
            pl.BlockSpec((BN_N, 128), lambda i: (i, 0)),
            pl.BlockSpec((BN_N, 16), lambda i: (i, 0)),
            pl.BlockSpec((128, 128), lambda i: (0, 0)),
            pl.BlockSpec((16, 128), lambda i: (0, 0)),
        ],
        out_specs=pl.BlockSpec((BN_N, 128), lambda i: (i, 0)),
        out_shape=jax.ShapeDtypeStruct((N, H), jnp.float32),
    )(x, pe, w1, w2)


def _qkv_body(y_ref, s_ref, c_ref, w_ref, q_ref, k_ref, v_ref):
    x = y_ref[...] * s_ref[...] + c_ref[...]
    t = _dotd(x, w_ref[...])
    q_ref[...] = t[:, 0:128]
    k_ref[...] = t[:, 128:256]
    v_ref[...] = t[:, 256:384]


def _qkv(y, s, c, w):
    g = N // BN_N
    sh = jax.ShapeDtypeStruct((N, H), jnp.float32)
    return pl.pallas_call(
        _qkv_body,
        grid=(g,),
        in_specs=[
            pl.BlockSpec((BN_N, 128), lambda i: (i, 0)),
            pl.BlockSpec((1, 128), lambda i: (0, 0)),
            pl.BlockSpec((1, 128), lambda i: (0, 0)),
            pl.BlockSpec((128, 384), lambda i: (0, 0)),
        ],
        out_specs=[pl.BlockSpec((BN_N, 128), lambda i: (i, 0))] * 3,
        out_shape=(sh, sh, sh),
    )(y, s, c, w)


def _eproj_body(ea_ref, wemb_ref, we_ref, o1_ref, o2_ref):
    e1 = _dotd(ea_ref[...], wemb_ref[...])
    o1_ref[...] = e1
    o2_ref[...] = _dotd(e1, we_ref[...]) * 0.25


def _eproj(ea, wemb, we):
    g = E // BN_E
    sh = jax.ShapeDtypeStruct((E, 128), jnp.float32)
    return pl.pallas_call(
        _eproj_body,
        grid=(g,),
        in_specs=[
            pl.BlockSpec((BN_E, 16), lambda i: (i, 0)),
            pl.BlockSpec((16, 128), lambda i: (0, 0)),
            pl.BlockSpec((128, 128), lambda i: (0, 0)),
        ],
        out_specs=[pl.BlockSpec((BN_E, 128), lambda i: (i, 0))] * 2,
        out_shape=(sh, sh),
    )(ea, wemb, we)


def _ee_body(u_ref, s_ref, c_ref, we_ref, o_ref):
    e = u_ref[...] * s_ref[...] + c_ref[...]
    o_ref[...] = _dotd(e, we_ref[...]) * 0.25


def _ee(u, s, c, we):
    g = E // BN_E
    return pl.pallas_call(
        _ee_body,
        grid=(g,),
        in_specs=[
            pl.BlockSpec((BN_E, 128), lambda i: (i, 0)),
            pl.BlockSpec((1, 128), lambda i: (0, 0)),
            pl.BlockSpec((1, 128), lambda i: (0, 0)),
            pl.BlockSpec((128, 128), lambda i: (0, 0)),
        ],
        out_specs=pl.BlockSpec((BN_E, 128), lambda i: (i, 0)),
        out_shape=jax.ShapeDtypeStruct((E, 128), jnp.float32),
    )(u, s, c, we)


def _pass_p_body(att_ref, up_ref, s_ref, c_ref, woe_ref, sel_ref,
                 u_ref, sc_ref, st_ref):
    i = pl.program_id(0)
    att = att_ref[...]
    sc_ref[...] = _dot(att, sel_ref[...])
    u = up_ref[...] * s_ref[...] + c_ref[...] + _dotd(att, woe_ref[...])
    u_ref[...] = u

    @pl.when(i == 0)
    def _():
        st_ref[...] = jnp.zeros_like(st_ref)

    st_ref[...] += jnp.concatenate(
        [jnp.sum(u, axis=0).reshape(1, 128),
         jnp.sum(u * u, axis=0).reshape(1, 128),
         jnp.zeros((6, 128), jnp.float32)], axis=0)


def _pass_p1_body(att_ref, ea_ref, we_ref, woe_ref, sel_ref,
                  u_ref, sc_ref, st_ref):
    i = pl.program_id(0)
    att = att_ref[...]
    sc_ref[...] = _dot(att, sel_ref[...])
    u = _dotd(ea_ref[...], we_ref[...]) + _dotd(att, woe_ref[...])
    u_ref[...] = u

    @pl.when(i == 0)
    def _():
        st_ref[...] = jnp.zeros_like(st_ref)

    st_ref[...] += jnp.concatenate(
        [jnp.sum(u, axis=0).reshape(1, 128),
         jnp.sum(u * u, axis=0).reshape(1, 128),
         jnp.zeros((6, 128), jnp.float32)], axis=0)


def _pass_p4_body(att_ref, sel_ref, sc_ref):
    sc_ref[...] = _dot(att_ref[...], sel_ref[...])


_SEL = None


def _sel():
    global _SEL
    if _SEL is None:
        s = np.zeros((128, 8), np.float32)
        for d in range(128):
            s[d, d // 16] = 1.0
        _SEL = jnp.asarray(s)
    return _SEL


def _pass_p(att, u_prev, s, c, woe, first, we=None):
    g = E // BN_E
    outs = (jax.ShapeDtypeStruct((E, 128), jnp.float32),
            jax.ShapeDtypeStruct((E, 8), jnp.float32),
            jax.ShapeDtypeStruct((8, 128), jnp.float32))
    out_specs = [pl.BlockSpec((BN_E, 128), lambda i: (i, 0)),
                 pl.BlockSpec((BN_E, 8), lambda i: (i, 0)),
                 pl.BlockSpec((8, 128), lambda i: (0, 0))]
    if first:
        return pl.pallas_call(
            _pass_p1_body,
            grid=(g,),
            in_specs=[
                pl.BlockSpec((BN_E, 128), lambda i: (i, 0)),
                pl.BlockSpec((BN_E, 16), lambda i: (i, 0)),
                pl.BlockSpec((16, 128), lambda i: (0, 0)),
                pl.BlockSpec((128, 128), lambda i: (0, 0)),
                pl.BlockSpec((128, 8), lambda i: (0, 0)),
            ],
            out_specs=out_specs,
            out_shape=outs,
        )(att, u_prev, we, woe, _sel())
    return pl.pallas_call(
        _pass_p_body,
        grid=(g,),
        in_specs=[
            pl.BlockSpec((BN_E, 128), lambda i: (i, 0)),
            pl.BlockSpec((BN_E, 128), lambda i: (i, 0)),
            pl.BlockSpec((1, 128), lambda i: (0, 0)),
            pl.BlockSpec((1, 128), lambda i: (0, 0)),
            pl.BlockSpec((128, 128), lambda i: (0, 0)),
            pl.BlockSpec((128, 8), lambda i: (0, 0)),
        ],
        out_specs=out_specs,
        out_shape=outs,
    )(att, u_prev, s, c, woe, _sel())


def _pass_p4(att):
    g = E // BN_E
    return pl.pallas_call(
        _pass_p4_body,
        grid=(g,),
        in_specs=[
            pl.BlockSpec((BN_E, 128), lambda i: (i, 0)),
            pl.BlockSpec((128, 8), lambda i: (0, 0)),
        ],
        out_specs=pl.BlockSpec((BN_E, 8), lambda i: (i, 0)),
        out_shape=jax.ShapeDtypeStruct((E, 8), jnp.float32),
    )(att, _sel())


def _node_a_body(y_ref, agg_ref, agg2_ref, s_ref, c_ref, wo_ref,
                 w_ref, st_ref):
    i = pl.program_id(0)
    w = (y_ref[...] * s_ref[...] + c_ref[...] +
         _dotd(agg_ref[...] + agg2_ref[...], wo_ref[...]))
    w_ref[...] = w

    @pl.when(i == 0)
    def _():
        st_ref[...] = jnp.zeros_like(st_ref)

    st_ref[...] += jnp.concatenate(
        [jnp.sum(w, axis=0).reshape(1, 128),
         jnp.sum(w * w, axis=0).reshape(1, 128),
         jnp.zeros((6, 128), jnp.float32)], axis=0)


def _node_a(y, agg, s, c, wo):
    g = N // BN_N
    return pl.pallas_call(
        _node_a_body,
        grid=(g,),
        in_specs=[
            pl.BlockSpec((BN_N, 128), lambda i: (i, 0)),
            pl.BlockSpec((BN_N, 128), lambda i: (i, 0)),
            pl.BlockSpec((BN_N, 128), lambda i: (i + N // BN_N, 0)),
            pl.BlockSpec((1, 128), lambda i: (0, 0)),
            pl.BlockSpec((1, 128), lambda i: (0, 0)),
            pl.BlockSpec((128, 128), lambda i: (0, 0)),
        ],
        out_specs=[pl.BlockSpec((BN_N, 128), lambda i: (i, 0)),
                   pl.BlockSpec((8, 128), lambda i: (0, 0))],
        out_shape=(jax.ShapeDtypeStruct((N, 128), jnp.float32),
                   jax.ShapeDtypeStruct((8, 128), jnp.float32)),
    )(y, agg, agg, s, c, wo)


def _node_b_body(w_ref, s_ref, c_ref, w1_ref, b1_ref, w2_ref, b2_ref,
                 y_ref, st_ref):
    i = pl.program_id(0)
    x1 = w_ref[...] * s_ref[...] + c_ref[...]
    h = _dotd(jnp.maximum(_dotd(x1, w1_ref[...]) + b1_ref[...], 0.0), w2_ref[...]) + b2_ref[...]
    y = x1 + h
    y_ref[...] = y

    @pl.when(i == 0)
    def _():
        st_ref[...] = jnp.zeros_like(st_ref)

    st_ref[...] += jnp.concatenate(
        [jnp.sum(y, axis=0).reshape(1, 128),
         jnp.sum(y * y, axis=0).reshape(1, 128),
         jnp.zeros((6, 128), jnp.float32)], axis=0)


def _node_b(w, s, c, w1, b1, w2, b2):
    g = N // BN_N
    return pl.pallas_call(
        _node_b_body,
        grid=(g,),
        in_specs=[
            pl.BlockSpec((BN_N, 128), lambda i: (i, 0)),
            pl.BlockSpec((1, 128), lambda i: (0, 0)),
            pl.BlockSpec((1, 128), lambda i: (0, 0)),
            pl.BlockSpec((128, FF), lambda i: (0, 0)),
            pl.BlockSpec((1, FF), lambda i: (0, 0)),
            pl.BlockSpec((FF, 128), lambda i: (0, 0)),
            pl.BlockSpec((1, 128), lambda i: (0, 0)),
        ],
        out_specs=[pl.BlockSpec((BN_N, 128), lambda i: (i, 0)),
                   pl.BlockSpec((8, 128), lambda i: (0, 0))],
        out_shape=(jax.ShapeDtypeStruct((N, 128), jnp.float32),
                   jax.ShapeDtypeStruct((8, 128), jnp.float32)),
    )(w, s, c, w1, b1, w2, b2)


def _node_b4_body(w_ref, batch_ref, s_ref, c_ref, w1_ref, b1_ref,
                  w2_ref, b2_ref, pool_ref, st_ref):
    i = pl.program_id(0)
    x1 = w_ref[...] * s_ref[...] + c_ref[...]
    h = _dotd(jnp.maximum(_dotd(x1, w1_ref[...]) + b1_ref[...], 0.0), w2_ref[...]) + b2_ref[...]
    y = x1 + h

    @pl.when(i == 0)
    def _():
        st_ref[...] = jnp.zeros_like(st_ref)
        pool_ref[...] = jnp.zeros_like(pool_ref)

    st_ref[...] += jnp.concatenate(
        [jnp.sum(y, axis=0).reshape(1, 128),
         jnp.sum(y * y, axis=0).reshape(1, 128),
         jnp.zeros((6, 128), jnp.float32)], axis=0)

    gids = jax.lax.broadcasted_iota(jnp.int32, (NG, BN_N), 0)
    onehot = (gids == batch_ref[0]).astype(jnp.float32)
    pool_ref[...] += jax.lax.dot_general(
        onehot, y, (((1,), (0,)), ((), ())),
        precision=jax.lax.Precision.HIGHEST,
        preferred_element_type=jnp.float32)


def _node_b4(w, batch2d, s, c, w1, b1, w2, b2):
    g = N // BN_N
    return pl.pallas_call(
        _node_b4_body,
        grid=(g,),
        in_specs=[
            pl.BlockSpec((BN_N, 128), lambda i: (i, 0)),
            pl.BlockSpec((1, 1, BN_N), lambda i: (i, 0, 0)),
            pl.BlockSpec((1, 128), lambda i: (0, 0)),
            pl.BlockSpec((1, 128), lambda i: (0, 0)),
            pl.BlockSpec((128, FF), lambda i: (0, 0)),
            pl.BlockSpec((1, FF), lambda i: (0, 0)),
            pl.BlockSpec((FF, 128), lambda i: (0, 0)),
            pl.BlockSpec((1, 128), lambda i: (0, 0)),
        ],
        out_specs=[pl.BlockSpec((NG, 128), lambda i: (0, 0)),
                   pl.BlockSpec((8, 128), lambda i: (0, 0))],
        out_shape=(jax.ShapeDtypeStruct((NG, 128), jnp.float32),
                   jax.ShapeDtypeStruct((8, 128), jnp.float32)),
    )(w, batch2d, s, c, w1, b1, w2, b2)


def _heads_body(py_ref, cnt_ref, s_ref, c_ref,
                mw1, mb1, mw2, mb2, lw1, lb1, lw2, lb2,
                mu_ref, lv_ref):
    p = py_ref[...] * s_ref[...] + cnt_ref[...] * c_ref[...]
    hmu = jnp.maximum(_dotd(p, mw1[...]) + mb1[...], 0.0)
    mu_ref[...] = _dotd(hmu, mw2[...]) + mb2[...]
    hlv = jnp.maximum(_dotd(p, lw1[...]) + lb1[...], 0.0)
    lv_ref[...] = _dotd(hlv, lw2[...]) + lb2[...]


def _heads(pool_y, counts, s, c, params):
    return pl.pallas_call(
        _heads_body,
        out_shape=(jax.ShapeDtypeStruct((NG, 1), jnp.float32),
                   jax.ShapeDtypeStruct((NG, 1), jnp.float32)),
    )(pool_y, counts, s, c,
      params["mu_W1"], params["mu_b1"].reshape(1, H),
      params["mu_W2"], params["mu_b2"].reshape(1, 1),
      params["lv_W1"], params["lv_b1"].reshape(1, H),
      params["lv_W2"], params["lv_b2"].reshape(1, 1))


from jax import lax
from jax.experimental.pallas import tpu_sc as plsc

NW = 32
EPW = E // NW
CH = 200
NCH = EPW // CH
NPW = 313
NPAD = 320
CH_S = 64

_mesh = plsc.VectorSubcoreMesh(core_axis_name="c", subcore_axis_name="s")
_SC_PARAMS = pltpu.CompilerParams(needs_layout_passes=False)


def _wid():
    return lax.axis_index("s") * 2 + lax.axis_index("c")


def _iota16():
    return lax.broadcasted_iota(jnp.int32, (16,), 0)


def _sget(ref, i):
    base = (i // 16) * 16
    vec = ref[pl.ds(base, 16)]
    lane = _iota16()
    sel = lane == jnp.full((16,), i - base, jnp.int32)
    return jnp.max(jnp.where(sel, vec, jnp.full((16,), -2147483648,
                                                jnp.int32)))


def _splat(x):
    return jnp.full((16,), x, jnp.float32)


def _splat_i(x):
    return jnp.full((16,), x, jnp.int32)


@functools.partial(
    pl.kernel, mesh=_mesh, compiler_params=_SC_PARAMS,
    out_type=(jax.ShapeDtypeStruct((E, 128), jnp.float32),
              jax.ShapeDtypeStruct((E, 128), jnp.float32)),
    scratch_types=[
        pltpu.VMEM((104,), jnp.int32),
        pltpu.VMEM((96,), jnp.int32),
        pltpu.VMEM((CH, 128), jnp.float32),
        pltpu.VMEM((CH, 128), jnp.float32),
        pltpu.SemaphoreType.DMA,
    ],
)
def _k_perm(t1_hbm, t2_hbm, perm_hbm, o1_hbm, o2_hbm, ia, ib, b1, b2, sem):
    w = _wid()

    def chunk(i, _):
        base = w * EPW + i * CH
        pltpu.sync_copy(perm_hbm.at[pl.ds(base, 104)], ia)
        pltpu.sync_copy(perm_hbm.at[pl.ds(base + 104, 96)], ib)
        c1 = pltpu.async_copy(t1_hbm.at[ia], b1.at[pl.ds(0, 104)], sem)
        c2 = pltpu.async_copy(t1_hbm.at[ib], b1.at[pl.ds(104, 96)], sem)
        c3 = pltpu.async_copy(t2_hbm.at[ia], b2.at[pl.ds(0, 104)], sem)
        c4 = pltpu.async_copy(t2_hbm.at[ib], b2.at[pl.ds(104, 96)], sem)
        c1.wait()
        c2.wait()
        c3.wait()
        c4.wait()
        pltpu.sync_copy(b1, o1_hbm.at[pl.ds(base, CH)])
        pltpu.sync_copy(b2, o2_hbm.at[pl.ds(base, CH)])
        return 0

    lax.fori_loop(0, NCH, chunk, 0)


@functools.partial(
    pl.kernel, mesh=_mesh, compiler_params=_SC_PARAMS,
    out_type=jax.ShapeDtypeStruct((E, 128), jnp.float32),
    scratch_types=[
        pltpu.VMEM((104,), jnp.int32),
        pltpu.VMEM((96,), jnp.int32),
        pltpu.VMEM((104,), jnp.int32),
        pltpu.VMEM((96,), jnp.int32),
        pltpu.VMEM((CH, 128), jnp.float32),
        pltpu.VMEM((CH, 128), jnp.float32),
        pltpu.VMEM((CH, 128), jnp.float32),
        pltpu.SemaphoreType.DMA,
    ],
)
def _k_att(q_hbm, k_hbm, ee_hbm, dst_hbm, src_hbm, out_hbm,
           da, db, sa, sb_, qb, kb, eb, sem):
    w = _wid()

    def chunk(i, _):
        base = w * EPW + i * CH
        pltpu.sync_copy(dst_hbm.at[pl.ds(base, 104)], da)
        pltpu.sync_copy(dst_hbm.at[pl.ds(base + 104, 96)], db)
        pltpu.sync_copy(src_hbm.at[pl.ds(base, 104)], sa)
        pltpu.sync_copy(src_hbm.at[pl.ds(base + 104, 96)], sb_)
        c1 = pltpu.async_copy(q_hbm.at[da], qb.at[pl.ds(0, 104)], sem)
        c2 = pltpu.async_copy(q_hbm.at[db], qb.at[pl.ds(104, 96)], sem)
        c3 = pltpu.async_copy(k_hbm.at[sa], kb.at[pl.ds(0, 104)], sem)
        c4 = pltpu.async_copy(k_hbm.at[sb_], kb.at[pl.ds(104, 96)], sem)
        pltpu.sync_copy(ee_hbm.at[pl.ds(base, CH)], eb)
        c1.wait()
        c2.wait()
        c3.wait()
        c4.wait()

        def edge(e2, _):
            for j in range(8):
                sl = pl.ds(j * 16, 16)
                eb[e2, sl] = qb[e2, sl] * kb[e2, sl] * eb[e2, sl]
            return 0

        lax.fori_loop(0, CH, edge, 0)
        pltpu.sync_copy(eb, out_hbm.at[pl.ds(base, CH)])
        return 0

    lax.fori_loop(0, NCH, chunk, 0)


@functools.partial(
    pl.kernel, mesh=_mesh, compiler_params=_SC_PARAMS,
    out_type=jax.ShapeDtypeStruct((NW * NPAD * 128,), jnp.float32),
    scratch_types=[
        pltpu.VMEM((10008,), jnp.int32),
        pltpu.VMEM((CH_S * 8,), jnp.float32),
        pltpu.VMEM((16,), jnp.float32),
        pltpu.VMEM((64 * 128,), jnp.float32),
        pltpu.SemaphoreType.DMA,
    ],
)
def _k_soft(rs_hbm, scf_hbm, out_hbm, rsb, sb, tmp, msw, sem):
    w = _wid()
    pltpu.sync_copy(rs_hbm, rsb)
    lane = _iota16()
    half = lax.shift_right_logical(lane, _splat_i(3))
    swap_idx = lax.bitwise_xor(lane, _splat_i(8))
    neg = _splat(-1e30)

    def node(nl, _):
        n = w * NPW + nl
        e_lo = nl * 16
        e_hi = e_lo + 16
        ab0 = (e_lo // 8) * 8
        nchunk = (e_hi - ab0 + CH_S - 1) // CH_S

        def chunk(ci, carry):
            m_dup, den = carry
            ab = ab0 + ci * CH_S
            pltpu.sync_copy(scf_hbm.at[pl.ds(ab * 8, CH_S * 8)], sb)
            lo_v = jnp.full((16,), e_lo, jnp.int32)
            hi_v = jnp.full((16,), e_hi, jnp.int32)

            def pair(ip, carry2):
                m_dup2, den2 = carry2
                glob = jnp.full((16,), ab + 2 * ip, jnp.int32) + half
                ok = jnp.logical_and(glob >= lo_v, glob < hi_v)
                s01 = plsc.load_gather(sb, [_splat_i(2 * ip * 8) + lane])
                s01 = jnp.where(ok, s01, neg)
                m1 = jnp.maximum(m_dup2, s01)
                tmp[...] = m1
                msw_sw = plsc.load_gather(tmp, [swap_idx])
                m_new = jnp.maximum(m1, msw_sw)
                c = jnp.exp(m_dup2 - m_new)
                ex = jnp.exp(s01 - m_new)
                den3 = den2 * c + ex
                return (m_new, den3)

            return lax.fori_loop(0, CH_S // 2, pair, (m_dup, den))

        m_dup, den = lax.fori_loop(0, nchunk, chunk, (neg, _splat(0.0)))
        tmp[...] = den
        den_t = den + plsc.load_gather(tmp, [swap_idx])
        out_v = jnp.where(half < _splat_i(1), m_dup, den_t)
        slot = nl % 64
        msw[pl.ds(slot * 128, 16)] = out_v

        @pl.when(jnp.logical_or(slot == 63, nl == NPW - 1))
        def _():
            wi = nl // 64
            pltpu.sync_copy(msw,
                            out_hbm.at[pl.ds((w * NPAD + wi * 64) * 128,
                                             64 * 128)])
        return 0

    lax.fori_loop(0, NPW, node, 0)


@functools.partial(
    pl.kernel, mesh=_mesh, compiler_params=_SC_PARAMS,
    out_type=jax.ShapeDtypeStruct((2 * N, 128), jnp.float32),
    scratch_types=[
        pltpu.VMEM((104,), jnp.int32),
        pltpu.VMEM((96,), jnp.int32),
        pltpu.VMEM((104,), jnp.int32),
        pltpu.VMEM((96,), jnp.int32),
        pltpu.VMEM((104,), jnp.int32),
        pltpu.VMEM((96,), jnp.int32),
        pltpu.VMEM((CH, 128), jnp.float32),
        pltpu.VMEM((CH, 128), jnp.float32),
        pltpu.VMEM((CH * 8,), jnp.float32),
        pltpu.VMEM((16,), jnp.float32),
        pltpu.VMEM((128, 128), jnp.float32),
        pltpu.VMEM_SHARED((N, 128), jnp.float32),
        pltpu.SemaphoreType.DMA,
    ],
)
def _k_wagg(v_hbm, ms_hbm, scf_hbm, dst_hbm, src_hbm, mi_hbm,
            out_hbm, da, db, sa, sb_, ma, mb, vb, msb, sb, tmp, zb,
            aggsh, sem):
    w = _wid()
    tile = lax.axis_index("s")
    lane = _iota16()
    half = lax.shift_right_logical(lane, _splat_i(3))
    lane8 = lax.bitwise_and(lane, _splat_i(7))
    e8 = _splat_i(8)

    def zrow(r, _):
        for jj in range(8):
            zb[r, pl.ds(jj * 16, 16)] = jnp.zeros((16,), jnp.float32)
        return 0

    lax.fori_loop(0, 128, zrow, 0)

    def zcp(i2, _):
        @pl.when(lax.rem(i2, 16) == tile)
        def _():
            pltpu.sync_copy(zb, aggsh.at[pl.ds(i2 * 128, 128)])
        return 0

    lax.fori_loop(0, 78, zcp, 0)

    @pl.when(tile == 0)
    def _():
        pltpu.sync_copy(zb.at[pl.ds(0, 16)], aggsh.at[pl.ds(9984, 16)])

    plsc.subcore_barrier()

    def chunk(i, _):
        base = w * EPW + i * CH
        pltpu.sync_copy(dst_hbm.at[pl.ds(base, 104)], da)
        pltpu.sync_copy(dst_hbm.at[pl.ds(base + 104, 96)], db)
        pltpu.sync_copy(src_hbm.at[pl.ds(base, 104)], sa)
        pltpu.sync_copy(src_hbm.at[pl.ds(base + 104, 96)], sb_)
        pltpu.sync_copy(mi_hbm.at[pl.ds(base, 104)], ma)
        pltpu.sync_copy(mi_hbm.at[pl.ds(base + 104, 96)], mb)
        c1 = pltpu.async_copy(v_hbm.at[sa], vb.at[pl.ds(0, 104)], sem)
        c2 = pltpu.async_copy(v_hbm.at[sb_], vb.at[pl.ds(104, 96)], sem)
        c3 = pltpu.async_copy(ms_hbm.at[ma], msb.at[pl.ds(0, 104)], sem)
        c4 = pltpu.async_copy(ms_hbm.at[mb], msb.at[pl.ds(104, 96)], sem)
        pltpu.sync_copy(scf_hbm.at[pl.ds(base * 8, CH * 8)], sb)
        c1.wait()
        c2.wait()
        c3.wait()
        c4.wait()

        def pair(ip, _):
            s01 = plsc.load_gather(sb, [_splat_i(2 * ip * 8) + lane])
            rowi = _splat_i(2 * ip) + half
            sm01 = plsc.load_gather(msb, [rowi, lane8])
            dn01 = plsc.load_gather(msb, [rowi, lane8 + e8])
            al01 = jnp.exp(s01 - sm01) / (dn01 + _splat(1e-9))
            tmp[...] = al01
            for jj in range(8):
                sl = pl.ds(jj * 16, 16)
                a0 = plsc.load_gather(tmp, [_splat_i(jj)])
                a1 = plsc.load_gather(tmp, [_splat_i(jj + 8)])
                vb[2 * ip, sl] = vb[2 * ip, sl] * a0
                vb[2 * ip + 1, sl] = vb[2 * ip + 1, sl] * a1
            return 0

        lax.fori_loop(0, CH // 2, pair, 0)
        pltpu.sync_copy(vb.at[pl.ds(0, 104)], aggsh.at[da], add=True)
        pltpu.sync_copy(vb.at[pl.ds(104, 96)], aggsh.at[db], add=True)
        return 0

    lax.fori_loop(0, NCH, chunk, 0)
    plsc.subcore_barrier()
    core = lax.axis_index("c")

    @pl.when(tile == 0)
    def _():
        pltpu.sync_copy(aggsh, out_hbm.at[pl.ds(core * N, N)])


def _affine_from_stats(st, g, b):
    n = st.shape[1] if False else None
    return None


def _bn_affine(st, cnt, g, b):
    m = st[0] / cnt
    var = st[1] / cnt - m * m
    s = g / jnp.sqrt(var + 1e-5)
    return s, b - m * s


def _edge_softmax_agg(score, v, src_s, dst_s):
    m = jax.ops.segment_max(score, dst_s, num_segments=N)
    m = jnp.where(jnp.isfinite(m), m, 0.0)
    ex = jnp.exp(score - m[dst_s])
    den = jax.ops.segment_sum(ex, dst_s, num_segments=N)
    alpha = ex / (den[dst_s] + 1e-9)
    vv = v.reshape(N, NH, DH)
    agg = jax.ops.segment_sum(vv[src_s] * alpha[:, :, None], dst_s,
                              num_segments=N)
    return agg.reshape(N, H)


def _gather_att(q, k, ee, src_s, dst_s):
    return q[dst_s] * k[src_s] * ee


@jax.jit
def _run(x, edge_index, edge_attr, pe, batch, params):
    src = edge_index[0]
    dst = edge_index[1]
    perm = jnp.argsort(dst).astype(jnp.int32)
    dst_s = dst[perm]
    src_s = src[perm]

    rs = jnp.searchsorted(dst_s, jnp.arange(N + 1, dtype=jnp.int32))
    rs_pad = jnp.concatenate(
        [rs.astype(jnp.int32), jnp.full((7,), E, jnp.int32)])
    midx = dst_s + 7 * (dst_s // NPW)
    t1, t2 = _eproj(edge_attr, params["edge_emb"],
                    params["layers"][0]["WE"])
    e1_s, ee1_s = _k_perm(t1, t2, perm)
    batch2d = batch.reshape(N // BN_N, 1, BN_N)
    counts = (jnp.searchsorted(batch, jnp.arange(1, NG + 1), side="left") -
              jnp.searchsorted(batch, jnp.arange(NG), side="left"))
    counts = counts.astype(jnp.float32).reshape(NG, 1)

    zeros128 = jnp.zeros((1, 128), jnp.float32)
    z384 = jnp.zeros((1, 384), jnp.float32)

    h0 = _emb(x, pe, params["node_emb"], params["pe_emb"])

    layers = params["layers"]
    y = h0
    sy, cy = jnp.ones((128,)), jnp.zeros((128,))
    u = None
    su, cu = None, None

    for li in range(NL):
        p = layers[li]
        first = li == 0
        last = li == NL - 1

        wqkv = jnp.concatenate([p["WQ"], p["WK"], p["WV"]], axis=1)
        q, k, v = _qkv(y, sy.reshape(1, 128), cy.reshape(1, 128), wqkv)

        if first:
            ee = ee1_s
        else:
            ee = _ee(u, su.reshape(1, 128), cu.reshape(1, 128), p["WE"])

        att = _k_att(q, k, ee, dst_s, src_s)

        if last:
            score = _pass_p4(att)
        else:
            if first:
                ones = jnp.ones((1, 128), jnp.float32)
                zer = jnp.zeros((1, 128), jnp.float32)
                u, score, st_u = _pass_p(att, e1_s, ones, zer, p["WOe"],
                                         first=False)
            else:
                u, score, st_u = _pass_p(att, u, su.reshape(1, 128),
                                         cu.reshape(1, 128), p["WOe"],
                                         first=False)
            su, cu = _bn_affine(st_u, E, p["ge"], p["bee"])

        agg0 = _edge_softmax_agg(score, v, src_s, dst_s)
        agg = jnp.concatenate([agg0, jnp.zeros((N, 128), jnp.float32)], 0)

        w, st_w = _node_a(y, agg, sy.reshape(1, 128), cy.reshape(1, 128),
                          p["WO"])
        sw, cw = _bn_affine(st_w, N, p["g1"], p["be1"])
        if last:
            pool_y, st_y = _node_b4(w, batch2d, sw.reshape(1, 128),
                                    cw.reshape(1, 128), p["W1"],
                                    p["b1"].reshape(1, FF), p["W2"],
                                    p["b2"].reshape(1, 128))
        else:
            y, st_y = _node_b(w, sw.reshape(1, 128), cw.reshape(1, 128),
                              p["W1"], p["b1"].reshape(1, FF), p["W2"],
                              p["b2"].reshape(1, 128))
        sy, cy = _bn_affine(st_y, N, p["g2"], p["be2"])

    mu, lv = _heads(pool_y, counts, sy.reshape(1, 128), cy.reshape(1, 128),
                    params)
    return mu, lv


def kernel(x, edge_index, edge_attr, pe, batch, params):
    return _run(x, edge_index, edge_attr, pe, batch, params)
